# Initial kernel scaffold; baseline (speedup 1.0000x reference)
import jax
import jax.numpy as jnp
from jax.experimental import pallas as pl


def kernel(x, edge_index, edge_weight, flepe, W1, b1, We1, W2, b2, We2):
    raise NotImplementedError("write your pallas kernel here")


# speedup vs baseline: 6.3790x; 6.3790x over previous
import functools

import jax
import jax.numpy as jnp
from jax import lax
from jax.experimental import pallas as pl
from jax.experimental.pallas import tpu as pltpu
from jax.experimental.pallas import tpu_sc as plsc

NC = 2
NS = 16
NW = NC * NS
LANES = 16
CHUNK = 80


def _mesh():
  return plsc.VectorSubcoreMesh(
      core_axis_name="c", subcore_axis_name="s", num_cores=NC, num_subcores=NS)


def _zero_fill_2d(ref, rows, cols):
  z = jnp.zeros((LANES,), jnp.float32)

  def body(i, _):
    for j in range(cols // LANES):
      ref[i, pl.ds(j * LANES, LANES)] = z
    return 0

  lax.fori_loop(0, rows, body, 0, unroll=2)


def _make_segsum(E, N, DE):
  epw = E // NW
  assert epw * NW == E and epw % CHUNK == 0
  nch = epw // CHUNK
  nblk = N // CHUNK
  assert nblk * CHUNK == N
  mesh = _mesh()

  @functools.partial(
      pl.kernel,
      out_type=(jax.ShapeDtypeStruct((NC, N, DE), jnp.float32),
                jax.ShapeDtypeStruct((NC, N, DE), jnp.float32)),
      mesh=mesh,
      scratch_types=[
          pltpu.VMEM((CHUNK,), jnp.int32),
          pltpu.VMEM((CHUNK, DE), jnp.float32),
          pltpu.VMEM((CHUNK,), jnp.float32),
          pltpu.VMEM((CHUNK, DE), jnp.float32),
          pltpu.VMEM_SHARED((N, DE), jnp.float32),
          pltpu.VMEM_SHARED((N, DE), jnp.float32),
      ],
  )
  def k(dst_hbm, flepe_hbm, ew_hbm, fs_out, dg_out,
        idx_v, f_v, ew_v, w_v, acc_fs, acc_dg):
    c = lax.axis_index("c")
    s = lax.axis_index("s")
    _zero_fill_2d(f_v, CHUNK, DE)
    for i in range((nblk + NS - 1) // NS):
      b = s + i * NS

      @pl.when(b < nblk)
      def _():
        pltpu.sync_copy(f_v, acc_fs.at[pl.ds(b * CHUNK, CHUNK), :])
        pltpu.sync_copy(f_v, acc_dg.at[pl.ds(b * CHUNK, CHUNK), :])

    plsc.subcore_barrier()

    base0 = (c * NS + s) * epw

    def body(i, _):
      base = base0 + i * CHUNK
      pltpu.sync_copy(dst_hbm.at[pl.ds(base, CHUNK)], idx_v)
      pltpu.sync_copy(flepe_hbm.at[pl.ds(base, CHUNK), :], f_v)
      pltpu.sync_copy(ew_hbm.at[pl.ds(base, CHUNK)], ew_v)
      lane0 = lax.iota(jnp.int32, LANES) == 0
      zv = jnp.zeros((LANES,), jnp.float32)

      def wrow(g, _):
        ew16 = ew_v[pl.ds(g * LANES, LANES)]
        for l in range(LANES):
          cvec = jnp.zeros((LANES,), jnp.float32) + ew16[l]
          w_v[g * LANES + l, :] = jnp.where(lane0, cvec, zv)
        return 0

      lax.fori_loop(0, CHUNK // LANES, wrow, 0)
      pltpu.sync_copy(f_v, acc_fs.at[idx_v], add=True)
      pltpu.sync_copy(w_v, acc_dg.at[idx_v], add=True)
      return 0

    lax.fori_loop(0, nch, body, 0)
    plsc.subcore_barrier()
    for i in range((nblk + NS - 1) // NS):
      b = s + i * NS

      @pl.when(b < nblk)
      def _():
        r = b * CHUNK
        pltpu.sync_copy(acc_fs.at[pl.ds(r, CHUNK), :],
                        fs_out.at[c, pl.ds(r, CHUNK), :])
        pltpu.sync_copy(acc_dg.at[pl.ds(r, CHUNK), :],
                        dg_out.at[c, pl.ds(r, CHUNK), :])

  return k


def _make_spmm(E, N, D):
  epw = E // NW
  nch = epw // CHUNK
  nblk = N // CHUNK
  assert nblk * CHUNK == N
  mesh = _mesh()

  @functools.partial(
      pl.kernel,
      out_type=jax.ShapeDtypeStruct((NC, N, D), jnp.float32),
      mesh=mesh,
      scratch_types=[
          pltpu.VMEM((CHUNK,), jnp.int32),
          pltpu.VMEM((CHUNK,), jnp.int32),
          pltpu.VMEM((CHUNK,), jnp.float32),
          pltpu.VMEM((CHUNK, D), jnp.float32),
          pltpu.VMEM_SHARED((N, D), jnp.float32),
          pltpu.SemaphoreType.DMA,
      ],
  )
  def k(src_hbm, dst_hbm, ew_hbm, h_hbm, out_hbm,
        src_v, dst_v, ew_v, g_v, acc, sem):
    c = lax.axis_index("c")
    s = lax.axis_index("s")
    _zero_fill_2d(g_v, CHUNK, D)
    for i in range((nblk + NS - 1) // NS):
      b = s + i * NS

      @pl.when(b < nblk)
      def _():
        pltpu.sync_copy(g_v, acc.at[pl.ds(b * CHUNK, CHUNK), :])

    plsc.subcore_barrier()

    base0 = (c * NS + s) * epw

    def body(i, _):
      base = base0 + i * CHUNK
      pltpu.sync_copy(src_hbm.at[pl.ds(base, CHUNK)], src_v)
      pltpu.sync_copy(dst_hbm.at[pl.ds(base, CHUNK)], dst_v)
      pltpu.sync_copy(ew_hbm.at[pl.ds(base, CHUNK)], ew_v)
      pltpu.async_copy(h_hbm.at[src_v], g_v, sem).wait()

      def scale(g, _):
        ew16 = ew_v[pl.ds(g * LANES, LANES)]
        for l in range(LANES):
          cvec = jnp.zeros((LANES,), jnp.float32) + ew16[l]
          e = g * LANES + l
          for j in range(D // LANES):
            g_v[e, pl.ds(j * LANES, LANES)] = g_v[e, pl.ds(j * LANES, LANES)] * cvec
        return 0

      lax.fori_loop(0, CHUNK // LANES, scale, 0)
      pltpu.sync_copy(g_v, acc.at[dst_v], add=True)
      return 0

    lax.fori_loop(0, nch, body, 0)
    plsc.subcore_barrier()
    for i in range((nblk + NS - 1) // NS):
      b = s + i * NS

      @pl.when(b < nblk)
      def _():
        r = b * CHUNK
        pltpu.sync_copy(acc.at[pl.ds(r, CHUNK), :],
                        out_hbm.at[c, pl.ds(r, CHUNK), :])

  return k


_RB = 200


def _dinv_from(deg_ref):
  deg = deg_ref[0, :, 0:1] + deg_ref[1, :, 0:1]
  return jnp.where(deg > 0, lax.rsqrt(jnp.maximum(deg, 1e-12)), 0.0)


def _tc_h1(x, W1, degp):
  N, D_IN = x.shape
  DH = W1.shape[1]
  DE = degp.shape[2]

  def body(x_ref, w_ref, deg_ref, o_ref):
    dinv = _dinv_from(deg_ref)
    o_ref[...] = dinv * jnp.dot(x_ref[...], w_ref[...],
                                preferred_element_type=jnp.float32)

  return pl.pallas_call(
      body,
      grid=(N // _RB,),
      in_specs=[
          pl.BlockSpec((_RB, D_IN), lambda i: (i, 0)),
          pl.BlockSpec((D_IN, DH), lambda i: (0, 0)),
          pl.BlockSpec((NC, _RB, DE), lambda i: (0, i, 0)),
      ],
      out_specs=pl.BlockSpec((_RB, DH), lambda i: (i, 0)),
      out_shape=jax.ShapeDtypeStruct((N, DH), jnp.float32),
  )(x, W1, degp)


def _tc_mid(s1p, degp, fsp, We1, b1, W2):
  N = s1p.shape[1]
  DH = s1p.shape[2]
  DE = fsp.shape[2]
  DO = W2.shape[1]

  def body(s_ref, deg_ref, fs_ref, we_ref, b_ref, w2_ref, o_ref):
    dinv = _dinv_from(deg_ref)
    fs = fs_ref[0] + fs_ref[1]
    s1 = s_ref[0] + s_ref[1]
    out1 = jnp.maximum(
        dinv * s1 + jnp.dot(fs, we_ref[...], preferred_element_type=jnp.float32)
        + b_ref[...], 0.0)
    o_ref[...] = dinv * jnp.dot(out1, w2_ref[...],
                                preferred_element_type=jnp.float32)

  return pl.pallas_call(
      body,
      grid=(N // _RB,),
      in_specs=[
          pl.BlockSpec((NC, _RB, DH), lambda i: (0, i, 0)),
          pl.BlockSpec((NC, _RB, DE), lambda i: (0, i, 0)),
          pl.BlockSpec((NC, _RB, DE), lambda i: (0, i, 0)),
          pl.BlockSpec((DE, DH), lambda i: (0, 0)),
          pl.BlockSpec((1, DH), lambda i: (0, 0)),
          pl.BlockSpec((DH, DO), lambda i: (0, 0)),
      ],
      out_specs=pl.BlockSpec((_RB, DO), lambda i: (i, 0)),
      out_shape=jax.ShapeDtypeStruct((N, DO), jnp.float32),
  )(s1p, degp, fsp, We1, b1, W2)


def _tc_final(s2p, degp, fsp, We2, b2):
  N = s2p.shape[1]
  DO = s2p.shape[2]
  DE = fsp.shape[2]

  def body(s_ref, deg_ref, fs_ref, we_ref, b_ref, o_ref):
    dinv = _dinv_from(deg_ref)
    fs = fs_ref[0] + fs_ref[1]
    s2 = s_ref[0] + s_ref[1]
    o_ref[...] = (dinv * s2
                  + jnp.dot(fs, we_ref[...], preferred_element_type=jnp.float32)
                  + b_ref[...])

  return pl.pallas_call(
      body,
      grid=(N // _RB,),
      in_specs=[
          pl.BlockSpec((NC, _RB, DO), lambda i: (0, i, 0)),
          pl.BlockSpec((NC, _RB, DE), lambda i: (0, i, 0)),
          pl.BlockSpec((NC, _RB, DE), lambda i: (0, i, 0)),
          pl.BlockSpec((DE, DO), lambda i: (0, 0)),
          pl.BlockSpec((1, DO), lambda i: (0, 0)),
      ],
      out_specs=pl.BlockSpec((_RB, DO), lambda i: (i, 0)),
      out_shape=jax.ShapeDtypeStruct((N, DO), jnp.float32),
  )(s2p, degp, fsp, We2, b2)


def kernel(x, edge_index, edge_weight, flepe, W1, b1, We1, W2, b2, We2):
  N = x.shape[0]
  E = edge_index.shape[1]
  DE = flepe.shape[1]

  src = edge_index[0]
  dst = edge_index[1]
  b1r = b1.reshape(1, -1)
  b2r = b2.reshape(1, -1)

  segsum = _make_segsum(E, N, DE)
  fsp, degp = segsum(dst, flepe, edge_weight)

  spmm = _make_spmm(E, N, W1.shape[1])
  h1p = _tc_h1(x, W1, degp)
  s1p = spmm(src, dst, edge_weight, h1p)
  h2p = _tc_mid(s1p, degp, fsp, We1, b1r, W2)
  s2p = spmm(src, dst, edge_weight, h2p)
  return _tc_final(s2p, degp, fsp, We2, b2r)
